# baseline (speedup 1.0000x reference)
import functools

import jax
import jax.numpy as jnp
from jax import lax
from jax.experimental import pallas as pl
from jax.experimental.pallas import tpu as pltpu
from jax.experimental.pallas import tpu_sc as plsc

_L = 200
_DIM = 64
_PAD = 256
_NSUB = 16
_NCHUNK = _DIM // 16


@functools.partial(
    pl.kernel,
    out_type=jax.ShapeDtypeStruct((2 * _DIM,), jnp.float32),
    scratch_types=[
        pltpu.VMEM((16,), jnp.int32),
        pltpu.VMEM((16, _DIM), jnp.float32),
        pltpu.VMEM((_DIM,), jnp.float32),
        pltpu.VMEM_SHARED((_NSUB, _DIM), jnp.float32),
        pltpu.SemaphoreType.DMA,
    ],
    mesh=plsc.VectorSubcoreMesh(core_axis_name="c", subcore_axis_name="s"),
    compiler_params=pltpu.CompilerParams(use_tc_tiling_on_sc=False),
)
def _emb_agg(idx_hbm, table_hbm, out_hbm, idx_v, rows_v, vec_v, shared, sem):
    cid = lax.axis_index("c")
    sid = lax.axis_index("s")
    base = cid * _PAD + sid * 16

    pltpu.sync_copy(idx_hbm.at[pl.ds(base, 16)], idx_v)
    pltpu.async_copy(table_hbm.at[idx_v], rows_v, sem).wait()

    acc = [jnp.zeros((16,), jnp.float32) for _ in range(_NCHUNK)]
    for j in range(16):
        w = jnp.where(sid * 16 + j < _L, jnp.float32(1.0), jnp.float32(0.0))
        for c in range(_NCHUNK):
            acc[c] = acc[c] + rows_v[j, pl.ds(c * 16, 16)] * w
    for c in range(_NCHUNK):
        vec_v[pl.ds(c * 16, 16)] = acc[c]

    pltpu.sync_copy(vec_v, shared.at[sid])
    plsc.subcore_barrier()

    @pl.when(sid == 0)
    def _reduce():
        pltpu.sync_copy(shared, rows_v)
        tot = [jnp.zeros((16,), jnp.float32) for _ in range(_NCHUNK)]
        for r in range(_NSUB):
            for c in range(_NCHUNK):
                tot[c] = tot[c] + rows_v[r, pl.ds(c * 16, 16)]
        inv = jnp.float32(1.0 / _L)
        for c in range(_NCHUNK):
            vec_v[pl.ds(c * 16, 16)] = tot[c] * inv
        pltpu.sync_copy(vec_v, out_hbm.at[pl.ds(cid * _DIM, _DIM)])


def kernel(s1, s2, table):
    pad = jnp.zeros((_PAD - _L,), jnp.int32)
    idx = jnp.concatenate([s1.astype(jnp.int32), pad,
                           s2.astype(jnp.int32), pad])
    return _emb_agg(idx, table)
